# Initial kernel scaffold; baseline (speedup 1.0000x reference)
import jax
import jax.numpy as jnp
from jax.experimental import pallas as pl


def kernel(x, edge_index, batch, emb, W_in, b_in, p_in, W0, b0, p0, W1, b1, p1, att_w, att_b):
    raise NotImplementedError("write your pallas kernel here")


# speedup vs baseline: 2.7977x; 2.7977x over previous
import functools
import math

import jax
import jax.numpy as jnp
from jax import lax
from jax.experimental import pallas as pl
from jax.experimental.pallas import tpu as pltpu

N = 10000
E = 320000
L = 8
H = 128
G = 64
RATIO = 0.8

ROWS_BLK = 1000


def _combine_body(agge_ref, xw_ref, dinv_ref, nm_ref, b_ref, o_ref):
    dinv = dinv_ref[...]
    agg = dinv * agge_ref[...] + (dinv * dinv) * xw_ref[...]
    o_ref[...] = jnp.maximum((agg + b_ref[...]) * nm_ref[...], 0.0)


def _combine(aggE, xw, dinv, nm, b):
    grid = (N // ROWS_BLK,)
    return pl.pallas_call(
        _combine_body,
        grid=grid,
        in_specs=[
            pl.BlockSpec((ROWS_BLK, H), lambda i: (i, 0)),
            pl.BlockSpec((ROWS_BLK, H), lambda i: (i, 0)),
            pl.BlockSpec((ROWS_BLK, 1), lambda i: (i, 0)),
            pl.BlockSpec((ROWS_BLK, 1), lambda i: (i, 0)),
            pl.BlockSpec((1, H), lambda i: (0, 0)),
        ],
        out_specs=pl.BlockSpec((ROWS_BLK, H), lambda i: (i, 0)),
        out_shape=jax.ShapeDtypeStruct((N, H), jnp.float32),
    )(aggE, xw, dinv[:, None], nm[:, None], b[None, :])


def _topk_keep(score, batch, nm):
    onehot = (batch[None, :] == jnp.arange(G, dtype=jnp.int32)[:, None])
    onehot_f = onehot.astype(jnp.float32)
    counts = onehot_f @ nm
    k = jnp.ceil(RATIO * counts).astype(jnp.int32)

    u = lax.bitcast_convert_type(score, jnp.uint32)
    up = jnp.where(u >> 31 == 0, u | jnp.uint32(0x80000000), ~u)
    up = jnp.where(nm > 0, up, jnp.uint32(0))

    def body(_, cur):
        for bit in range(31, -1, -1):
            cand = cur | jnp.uint32(1 << bit)
            ge = (up[None, :] >= cand[:, None]) & onehot
            c = jnp.sum(ge, axis=1, dtype=jnp.int32)
            cur = jnp.where(c >= k, cand, cur)
        return cur
    M = body(None, jnp.zeros((G,), jnp.uint32))

    Mn = M[batch]
    gt = (up > Mn) & (nm > 0)
    eq = (up == Mn) & (nm > 0)
    gtc = onehot_f @ gt.astype(jnp.float32)
    r = k - gtc.astype(jnp.int32)
    eq_f = eq.astype(jnp.float32)
    csum = jnp.cumsum(eq_f) - eq_f
    eq_per_g = onehot_f @ eq_f
    start_eq = jnp.cumsum(eq_per_g) - eq_per_g
    tie_rank = csum - start_eq[batch]
    keep = gt | (eq & (tie_rank < r[batch].astype(jnp.float32)))
    return keep.astype(jnp.float32)


def _att_pool(h, batch, nm, att_w, att_b):
    onehot = (batch[None, :] == jnp.arange(G, dtype=jnp.int32)[:, None])
    onehot_f = onehot.astype(jnp.float32)
    gate = (h @ att_w)[:, 0] + att_b[0]
    gate_m = jnp.where(nm > 0, gate, -jnp.inf)
    gmax = jnp.max(jnp.where(onehot, gate_m[None, :], -jnp.inf), axis=1)
    gmax = jnp.where(jnp.isfinite(gmax), gmax, 0.0)
    e = jnp.exp(gate - gmax[batch]) * nm
    denom = onehot_f @ e
    alpha = e / jnp.maximum(denom[batch], 1e-16)
    return onehot_f @ (alpha[:, None] * h)


def kernel(x, edge_index, batch, emb, W_in, b_in, p_in, W0, b0, p0, W1, b1, p1, att_w, att_b):
    src, dst = edge_index[0], edge_index[1]
    tok = emb[x]
    m = (x != 0).astype(jnp.float32)
    node = (tok * m[..., None]).sum(1) / jnp.maximum(m.sum(1, keepdims=True), 1.0)

    nm = jnp.ones((N,), jnp.float32)
    h = node
    out = jnp.zeros((G, H), jnp.float32)
    for (W, b, p) in ((W_in, b_in, p_in), (W0, b0, p0), (W1, b1, p1)):
        deg_pre = jnp.zeros((N,), jnp.float32).at[dst].add(nm[src])
        deg = nm * (1.0 + deg_pre)
        dinv = jnp.where(deg > 0, lax.rsqrt(jnp.maximum(deg, 1e-12)), 0.0)
        xw = h @ W
        y = dinv[:, None] * xw
        aggE = jnp.zeros((N, H), jnp.float32).at[dst].add(y[src])
        h = _combine(aggE, xw, dinv, nm, b)
        score = h @ p / jnp.linalg.norm(p)
        keep = _topk_keep(jnp.where(nm > 0, score, -jnp.inf), batch, nm)
        h = h * jnp.tanh(score)[:, None] * keep[:, None]
        nm = keep
        out = out + _att_pool(h, batch, nm, att_w, att_b)
    return out


# speedup vs baseline: 19.8889x; 7.1090x over previous
import functools
import math

import jax
import jax.numpy as jnp
from jax import lax
from jax.experimental import pallas as pl
from jax.experimental.pallas import tpu as pltpu
from jax.experimental.pallas import tpu_sc as plsc

N = 10000
E = 320000
L = 8
H = 128
G = 64
RATIO = 0.8

ROWS_BLK = 1000

NC, NS = 2, 16
NW = NC * NS
CH = 128
NCHUNK = 79
EPW = NCHUNK * CH
EP = NW * EPW
NP = 10240
RPS = NP // NS

@functools.cache
def _sc_mesh():
    return plsc.VectorSubcoreMesh(
        core_axis_name="c", subcore_axis_name="s",
        num_cores=NC, num_subcores=NS)


@functools.cache
def _row_pass_kernel():
    return functools.partial(
        pl.kernel,
        out_type=jax.ShapeDtypeStruct((NC, NP, H), jnp.float32),
        mesh=_sc_mesh(),
        scratch_types=[
            pltpu.VMEM((NCHUNK, CH), jnp.int32),
            pltpu.VMEM((NCHUNK, CH), jnp.int32),
            pltpu.VMEM((CH, H), jnp.float32),
            pltpu.VMEM_SHARED((NP, H), jnp.float32),
            pltpu.SemaphoreType.DMA,
            pltpu.SemaphoreType.DMA,
        ],
    )(_sc_row_pass_body)


def _sc_row_pass_body(y_hbm, src_hbm, dst_hbm, zeros_hbm, out_hbm,
                 src_v, dst_v, rows_v, acc_sh, gsem, ssem):
    c = lax.axis_index("c")
    s = lax.axis_index("s")
    wid = c * NS + s
    pltpu.async_copy(zeros_hbm, acc_sh.at[pl.ds(s * RPS, RPS)], gsem).wait()
    pltpu.async_copy(src_hbm.at[wid], src_v, gsem).wait()
    pltpu.async_copy(dst_hbm.at[wid], dst_v, gsem).wait()
    plsc.subcore_barrier()

    def body(j, carry):
        pltpu.async_copy(y_hbm.at[src_v.at[j]], rows_v, gsem).wait()
        pltpu.async_copy(rows_v, acc_sh.at[dst_v.at[j]], ssem, add=True).wait()
        return carry

    lax.fori_loop(0, NCHUNK, body, 0)
    plsc.subcore_barrier()
    pltpu.async_copy(acc_sh.at[pl.ds(s * RPS, RPS)],
                     out_hbm.at[c].at[pl.ds(s * RPS, RPS)], ssem).wait()


@functools.cache
def _deg_pass_kernel():
    return functools.partial(
        pl.kernel,
        out_type=jax.ShapeDtypeStruct((NC, NP), jnp.float32),
        mesh=_sc_mesh(),
        scratch_types=[
            pltpu.VMEM((NCHUNK, CH), jnp.int32),
            pltpu.VMEM((NCHUNK, CH), jnp.int32),
            pltpu.VMEM((CH,), jnp.float32),
            pltpu.VMEM_SHARED((NP,), jnp.float32),
            pltpu.SemaphoreType.DMA,
            pltpu.SemaphoreType.DMA,
        ],
    )(_sc_deg_pass_body)


def _sc_deg_pass_body(nm_hbm, src_hbm, dst_hbm, zeros1_hbm, out_hbm,
                 src_v, dst_v, vals_v, deg_sh, gsem, ssem):
    c = lax.axis_index("c")
    s = lax.axis_index("s")
    wid = c * NS + s
    pltpu.async_copy(zeros1_hbm, deg_sh.at[pl.ds(s * RPS, RPS)], gsem).wait()
    pltpu.async_copy(src_hbm.at[wid], src_v, gsem).wait()
    pltpu.async_copy(dst_hbm.at[wid], dst_v, gsem).wait()
    plsc.subcore_barrier()

    def body(j, carry):
        pltpu.async_copy(nm_hbm.at[src_v.at[j]], vals_v, gsem).wait()
        pltpu.async_copy(vals_v, deg_sh.at[dst_v.at[j]], ssem, add=True).wait()
        return carry

    lax.fori_loop(0, NCHUNK, body, 0)
    plsc.subcore_barrier()
    pltpu.async_copy(deg_sh.at[pl.ds(s * RPS, RPS)],
                     out_hbm.at[c].at[pl.ds(s * RPS, RPS)], ssem).wait()


TBLK = (N * L) // CH
TPW = 20
NPB = CH // L


@functools.cache
def _emb_pass_kernel():
    return functools.partial(
        pl.kernel,
        out_type=jax.ShapeDtypeStruct((N, H), jnp.float32),
        mesh=_sc_mesh(),
        scratch_types=[
            pltpu.VMEM((TPW * CH,), jnp.int32),
            pltpu.VMEM((CH, H), jnp.float32),
            pltpu.VMEM((NPB, H), jnp.float32),
            pltpu.SemaphoreType.DMA,
            pltpu.SemaphoreType.DMA,
        ],
    )(_sc_emb_pass_body)


def _sc_emb_pass_body(xf_hbm, emb_hbm, out_hbm, tok_v, rows_v, node_v, gsem, ssem):
    c = lax.axis_index("c")
    s = lax.axis_index("s")
    wid = c * NS + s
    bbase = wid * TPW
    nblk = jnp.minimum(TBLK - bbase, TPW)
    pltpu.async_copy(xf_hbm.at[pl.ds(bbase * CH, TPW * CH)], tok_v, gsem).wait()

    def body(j, carry):
        pltpu.async_copy(emb_hbm.at[tok_v.at[pl.ds(j * CH, CH)]],
                         rows_v, gsem).wait()
        for n in range(NPB):
            for f in range(H // 16):
                sl = pl.ds(f * 16, 16)
                acc = rows_v[n * L, sl]
                for l in range(1, L):
                    acc = acc + rows_v[n * L + l, sl]
                node_v[n, sl] = acc
        pltpu.async_copy(node_v, out_hbm.at[pl.ds((bbase + j) * NPB, NPB)],
                         ssem).wait()
        return carry

    lax.fori_loop(0, nblk, body, 0)


def _combine_body(p_ref, xw_ref, dinv_ref, nm_ref, b_ref, o_ref):
    dinv = dinv_ref[...]
    agge = p_ref[0] + p_ref[1]
    agg = dinv * agge + (dinv * dinv) * xw_ref[...]
    o_ref[...] = jnp.maximum((agg + b_ref[...]) * nm_ref[...], 0.0)


def _combine(P, xw, dinv, nm, b):
    grid = (N // ROWS_BLK,)
    return pl.pallas_call(
        _combine_body,
        grid=grid,
        in_specs=[
            pl.BlockSpec((NC, ROWS_BLK, H), lambda i: (0, i, 0)),
            pl.BlockSpec((ROWS_BLK, H), lambda i: (i, 0)),
            pl.BlockSpec((ROWS_BLK, 1), lambda i: (i, 0)),
            pl.BlockSpec((ROWS_BLK, 1), lambda i: (i, 0)),
            pl.BlockSpec((1, H), lambda i: (0, 0)),
        ],
        out_specs=pl.BlockSpec((ROWS_BLK, H), lambda i: (i, 0)),
        out_shape=jax.ShapeDtypeStruct((N, H), jnp.float32),
    )(P, xw, dinv, nm, b[None, :])


def _k1_body(h_ref, t_ref, w_ref, degp_ref, nm_ref, xw_ref, y_ref, dinv_ref):
    nmv = nm_ref[...]
    deg = nmv * (1.0 + degp_ref[0] + degp_ref[1])
    dinv = jnp.where(deg > 0, lax.rsqrt(jnp.maximum(deg, 1e-12)), 0.0)
    hn = h_ref[...] * t_ref[...]
    xw = jnp.dot(hn, w_ref[...], preferred_element_type=jnp.float32)
    xw_ref[...] = xw
    y_ref[...] = dinv * xw
    dinv_ref[...] = dinv


def _k1(h, tcol, W, degP, nm):
    grid = (N // ROWS_BLK,)
    return pl.pallas_call(
        _k1_body,
        grid=grid,
        in_specs=[
            pl.BlockSpec((ROWS_BLK, H), lambda i: (i, 0)),
            pl.BlockSpec((ROWS_BLK, 1), lambda i: (i, 0)),
            pl.BlockSpec((H, H), lambda i: (0, 0)),
            pl.BlockSpec((NC, ROWS_BLK, 1), lambda i: (0, i, 0)),
            pl.BlockSpec((ROWS_BLK, 1), lambda i: (i, 0)),
        ],
        out_specs=[
            pl.BlockSpec((ROWS_BLK, H), lambda i: (i, 0)),
            pl.BlockSpec((ROWS_BLK, H), lambda i: (i, 0)),
            pl.BlockSpec((ROWS_BLK, 1), lambda i: (i, 0)),
        ],
        out_shape=[
            jax.ShapeDtypeStruct((N, H), jnp.float32),
            jax.ShapeDtypeStruct((N, H), jnp.float32),
            jax.ShapeDtypeStruct((N, 1), jnp.float32),
        ],
    )(h, tcol, W, degP[..., None], nm)


def _k1f_body(ns_ref, x_ref, w_ref, degp_ref, xw_ref, y_ref, dinv_ref):
    cnt = jnp.sum(jnp.where(x_ref[...] != 0, 1.0, 0.0), axis=1, keepdims=True)
    h = ns_ref[...] / jnp.maximum(cnt, 1.0)
    deg = 1.0 + degp_ref[0] + degp_ref[1]
    dinv = lax.rsqrt(jnp.maximum(deg, 1e-12))
    xw = jnp.dot(h, w_ref[...], preferred_element_type=jnp.float32)
    xw_ref[...] = xw
    y_ref[...] = dinv * xw
    dinv_ref[...] = dinv


def _k1f(nodesum, x, W, degP):
    grid = (N // ROWS_BLK,)
    return pl.pallas_call(
        _k1f_body,
        grid=grid,
        in_specs=[
            pl.BlockSpec((ROWS_BLK, H), lambda i: (i, 0)),
            pl.BlockSpec((ROWS_BLK, L), lambda i: (i, 0)),
            pl.BlockSpec((H, H), lambda i: (0, 0)),
            pl.BlockSpec((NC, ROWS_BLK, 1), lambda i: (0, i, 0)),
        ],
        out_specs=[
            pl.BlockSpec((ROWS_BLK, H), lambda i: (i, 0)),
            pl.BlockSpec((ROWS_BLK, H), lambda i: (i, 0)),
            pl.BlockSpec((ROWS_BLK, 1), lambda i: (i, 0)),
        ],
        out_shape=[
            jax.ShapeDtypeStruct((N, H), jnp.float32),
            jax.ShapeDtypeStruct((N, H), jnp.float32),
            jax.ShapeDtypeStruct((N, 1), jnp.float32),
        ],
    )(nodesum, x, W, degP[..., None])


_NT = (((1,), (1,)), ((), ()))


def _k2b_body(h_ref, brow_ref, nmrow_ref, pw_ref, aw_ref, ab_ref, out_ref,
              trow_ref, keep_ref, outn_ref):
    h = h_ref[...]
    pw = pw_ref[...]
    nmrow = nmrow_ref[...]
    score = (lax.dot_general(pw, h, _NT, preferred_element_type=jnp.float32)
             / jnp.sqrt(jnp.sum(pw * pw)))
    u = lax.bitcast_convert_type(score, jnp.uint32)
    up = jnp.where((u >> 31) == 0, u | jnp.uint32(0x80000000), ~u)
    up = jnp.where(nmrow > 0, up, jnp.uint32(0))

    grow = lax.broadcasted_iota(jnp.int32, (G, N), 0)
    ohb = brow_ref[...] == grow
    ohf = jnp.where(ohb, 1.0, 0.0)
    counts = jnp.sum(ohf * nmrow, axis=1, keepdims=True)
    kf = jnp.ceil(RATIO * counts)

    def bit_step(t, cur):
        bit = lax.shift_left(jnp.uint32(1), (jnp.int32(31) - t).astype(jnp.uint32))
        cand = cur | bit
        c = jnp.sum(jnp.where((up >= cand) & ohb, 1.0, 0.0), axis=1,
                    keepdims=True)
        return jnp.where(c >= kf, cand, cur)
    M = lax.fori_loop(0, 32, bit_step, jnp.zeros((G, 1), jnp.uint32))

    gt = (up > M) & ohb
    eq = (up == M) & ohb
    gtc = jnp.sum(jnp.where(gt, 1.0, 0.0), axis=1, keepdims=True)
    r = kf - gtc
    nidx = lax.broadcasted_iota(jnp.int32, (1, N), 1)

    def idx_step(t, cur):
        cand = cur | lax.shift_left(jnp.int32(1), jnp.int32(13) - t)
        c = jnp.sum(jnp.where(eq & (nidx < cand), 1.0, 0.0), axis=1,
                    keepdims=True)
        return jnp.where(c < r, cand, cur)
    T = lax.fori_loop(0, 14, idx_step, jnp.zeros((G, 1), jnp.int32))

    keep_ng = gt | (eq & (nidx <= T) & (r >= 1.0))
    keep = jnp.sum(jnp.where(keep_ng, 1.0, 0.0), axis=0, keepdims=True)

    trow = jnp.tanh(score) * keep
    ghdot = lax.dot_general(aw_ref[...], h, _NT,
                            preferred_element_type=jnp.float32)
    gate = trow * ghdot + ab_ref[0, 0]
    gate_ng = jnp.where(ohb & (keep > 0), gate, -jnp.inf)
    gmax = jnp.max(gate_ng, axis=1, keepdims=True)
    gmax = jnp.where(gmax == -jnp.inf, 0.0, gmax)
    gmax_row = jnp.sum(ohf * gmax, axis=0, keepdims=True)
    e = jnp.exp(gate - gmax_row) * keep
    denom = jnp.sum(ohf * e, axis=1, keepdims=True)
    denom_row = jnp.sum(ohf * denom, axis=0, keepdims=True)
    alpha = e / jnp.maximum(denom_row, 1e-16)
    wt = ohf * (alpha * trow)
    outn_ref[...] = out_ref[...] + jnp.dot(wt, h,
                                           preferred_element_type=jnp.float32)
    trow_ref[...] = trow
    keep_ref[...] = keep


def _k2b(h, brow, nmrow, p, att_w, att_b, out_prev):
    return pl.pallas_call(
        _k2b_body,
        out_shape=[
            jax.ShapeDtypeStruct((1, N), jnp.float32),
            jax.ShapeDtypeStruct((1, N), jnp.float32),
            jax.ShapeDtypeStruct((G, H), jnp.float32),
        ],
    )(h, brow, nmrow, p[None, :], att_w[:, 0][None, :], att_b[None, :],
      out_prev)


def _topk_keep(score, batch, nm):
    onehot = (batch[None, :] == jnp.arange(G, dtype=jnp.int32)[:, None])
    onehot_f = onehot.astype(jnp.float32)
    counts = onehot_f @ nm
    k = jnp.ceil(RATIO * counts).astype(jnp.int32)

    u = lax.bitcast_convert_type(score, jnp.uint32)
    up = jnp.where(u >> 31 == 0, u | jnp.uint32(0x80000000), ~u)
    up = jnp.where(nm > 0, up, jnp.uint32(0))

    def body(_, cur):
        for bit in range(31, -1, -1):
            cand = cur | jnp.uint32(1 << bit)
            ge = (up[None, :] >= cand[:, None]) & onehot
            c = jnp.sum(ge, axis=1, dtype=jnp.int32)
            cur = jnp.where(c >= k, cand, cur)
        return cur
    M = body(None, jnp.zeros((G,), jnp.uint32))

    Mn = M[batch]
    gt = (up > Mn) & (nm > 0)
    eq = (up == Mn) & (nm > 0)
    gtc = onehot_f @ gt.astype(jnp.float32)
    r = k - gtc.astype(jnp.int32)
    eq_f = eq.astype(jnp.float32)
    csum = jnp.cumsum(eq_f) - eq_f
    eq_per_g = onehot_f @ eq_f
    start_eq = jnp.cumsum(eq_per_g) - eq_per_g
    tie_rank = csum - start_eq[batch]
    keep = gt | (eq & (tie_rank < r[batch].astype(jnp.float32)))
    return keep.astype(jnp.float32)


def _att_pool(h, batch, nm, att_w, att_b):
    onehot = (batch[None, :] == jnp.arange(G, dtype=jnp.int32)[:, None])
    onehot_f = onehot.astype(jnp.float32)
    gate = (h @ att_w)[:, 0] + att_b[0]
    gate_m = jnp.where(nm > 0, gate, -jnp.inf)
    gmax = jnp.max(jnp.where(onehot, gate_m[None, :], -jnp.inf), axis=1)
    gmax = jnp.where(jnp.isfinite(gmax), gmax, 0.0)
    e = jnp.exp(gate - gmax[batch]) * nm
    denom = onehot_f @ e
    alpha = e / jnp.maximum(denom[batch], 1e-16)
    return onehot_f @ (alpha[:, None] * h)


def kernel(x, edge_index, batch, emb, W_in, b_in, p_in, W0, b0, p0, W1, b1, p1, att_w, att_b):
    src, dst = edge_index[0], edge_index[1]
    pad = EP - E
    srcR = jnp.concatenate([src, jnp.zeros((pad,), jnp.int32)]
                           ).reshape(NW, NCHUNK, CH)
    dstR = jnp.concatenate([dst, jnp.full((pad,), N, jnp.int32)]
                           ).reshape(NW, NCHUNK, CH)
    zrows = jnp.zeros((RPS, H), jnp.float32)
    z1 = jnp.zeros((RPS,), jnp.float32)
    xf = jnp.concatenate([x.reshape(-1),
                          jnp.zeros((NW * TPW * CH - N * L,), jnp.int32)])
    brow = batch[None, :]

    emb2 = emb.at[0].set(0.0)
    nodesum = _emb_pass_kernel()(xf, emb2)

    nmp = jnp.ones((NP,), jnp.float32)
    nm_col = jnp.ones((N, 1), jnp.float32)
    nm_row = jnp.ones((1, N), jnp.float32)
    t_col = None
    out = jnp.zeros((G, H), jnp.float32)
    h = nodesum
    for li, (W, b, p) in enumerate(((W_in, b_in, p_in), (W0, b0, p0), (W1, b1, p1))):
        degP = _deg_pass_kernel()(nmp, srcR, dstR, z1)[:, :N]
        if li == 0:
            xw, y, dinv = _k1f(h, x, W, degP)
        else:
            xw, y, dinv = _k1(h, t_col, W, degP, nm_col)
        P = _row_pass_kernel()(y, srcR, dstR, zrows)[:, :N]
        h = _combine(P, xw, dinv, nm_col, b)
        t_row, keep_row, out = _k2b(h, brow, nm_row, p, att_w, att_b, out)
        t_col = t_row.reshape(N, 1)
        nm_col = keep_row.reshape(N, 1)
        nm_row = keep_row
        nmp = jnp.concatenate([keep_row[0], jnp.zeros((NP - N,), jnp.float32)])
    return out
